# baseline (speedup 1.0000x reference)
import functools

import jax
import jax.numpy as jnp
from jax import lax
from jax.experimental import pallas as pl
from jax.experimental.pallas import tpu as pltpu
from jax.experimental.pallas import tpu_sc as plsc

NUM_FIELDS = 26
NUM_TOKEN_FIELDS = 20
NUM_FLOAT_FIELDS = 6
EMBED_DIM = 32
BATCH = 1024
VOCAB_PER_FIELD = 5000
TABLE_ROWS = 100000
NUM_PAIRS = (NUM_FIELDS * (NUM_FIELDS - 1)) // 2

L = 16
H = EMBED_DIM // L
NC, NS = 2, 16
NW = NC * NS
R = 4
CHUNKS = BATCH // R
CHUNKS_PER_W = CHUNKS // NW
COMBOS = NUM_TOKEN_FIELDS * NUM_FIELDS
IDX_PER_CHUNK = R * COMBOS
GATHER_DMA_ROWS = 104
GATHER_DMAS = IDX_PER_CHUNK // GATHER_DMA_ROWS
OUT_PER_CHUNK = R * NUM_PAIRS * EMBED_DIM
FX_PER_CHUNK = R * NUM_FLOAT_FIELDS * L


def _pair_index(i, j):
    return 25 * i - (i * (i - 1)) // 2 + (j - i - 1)


def _sc_body(tables, idx_all, fxr, ftf, out, idx_v, gbuf, obuf, fxv, ftv, sem):
    wid = lax.axis_index("s") * NC + lax.axis_index("c")
    pltpu.sync_copy(ftf, ftv)

    def chunk_body(g, carry):
        chunk = wid * CHUNKS_PER_W + g
        pltpu.sync_copy(idx_all.at[chunk], idx_v)
        pltpu.sync_copy(fxr.at[pl.ds(chunk * FX_PER_CHUNK, FX_PER_CHUNK)], fxv)
        copies = [
            pltpu.async_copy(
                tables.at[idx_v.at[pl.ds(d * GATHER_DMA_ROWS, GATHER_DMA_ROWS)]],
                gbuf.at[pl.ds(d * GATHER_DMA_ROWS, GATHER_DMA_ROWS)],
                sem,
            )
            for d in range(GATHER_DMAS)
        ]
        for c in copies:
            c.wait()

        def tt_i(i, c1):
            def tt_j(j, c2):
                p = 25 * i - (i * (i - 1)) // 2 + (j - i - 1)
                ca = i * NUM_FIELDS + j
                cb = j * NUM_FIELDS + i
                for r in range(R):
                    o_base = r * (NUM_PAIRS * EMBED_DIM) + p * EMBED_DIM
                    for h in range(H):
                        a = gbuf[r * COMBOS + ca, pl.ds(h * L, L)]
                        b = gbuf[r * COMBOS + cb, pl.ds(h * L, L)]
                        obuf[pl.ds(o_base + h * L, L)] = a * b
                return c2

            return lax.fori_loop(i + 1, NUM_TOKEN_FIELDS, tt_j, c1)

        lax.fori_loop(0, NUM_TOKEN_FIELDS, tt_i, 0)

        def tf_i(i, c1):
            p0 = 25 * i - (i * (i - 1)) // 2 + (19 - i)
            for k in range(NUM_FLOAT_FIELDS):
                p = p0 + k
                ca = i * NUM_FIELDS + NUM_TOKEN_FIELDS + k
                ft_base = (i * NUM_FLOAT_FIELDS + k) * EMBED_DIM
                for r in range(R):
                    o_base = r * (NUM_PAIRS * EMBED_DIM) + p * EMBED_DIM
                    fx = fxv[pl.ds((r * NUM_FLOAT_FIELDS + k) * L, L)]
                    for h in range(H):
                        ft = ftv[pl.ds(ft_base + h * L, L)]
                        a = gbuf[r * COMBOS + ca, pl.ds(h * L, L)]
                        obuf[pl.ds(o_base + h * L, L)] = a * (fx * ft)
            return c1

        lax.fori_loop(0, NUM_TOKEN_FIELDS, tf_i, 0)

        for a in range(NUM_FLOAT_FIELDS - 1):
            for b in range(a + 1, NUM_FLOAT_FIELDS):
                p = _pair_index(NUM_TOKEN_FIELDS + a, NUM_TOKEN_FIELDS + b)
                fta_base = ((NUM_TOKEN_FIELDS + b) * NUM_FLOAT_FIELDS + a) * EMBED_DIM
                ftb_base = ((NUM_TOKEN_FIELDS + a) * NUM_FLOAT_FIELDS + b) * EMBED_DIM
                for r in range(R):
                    o_base = r * (NUM_PAIRS * EMBED_DIM) + p * EMBED_DIM
                    fxa = fxv[pl.ds((r * NUM_FLOAT_FIELDS + a) * L, L)]
                    fxb = fxv[pl.ds((r * NUM_FLOAT_FIELDS + b) * L, L)]
                    for h in range(H):
                        fta = ftv[pl.ds(fta_base + h * L, L)]
                        ftb = ftv[pl.ds(ftb_base + h * L, L)]
                        obuf[pl.ds(o_base + h * L, L)] = (fxa * fta) * (fxb * ftb)

        pltpu.sync_copy(obuf, out.at[pl.ds(chunk * OUT_PER_CHUNK, OUT_PER_CHUNK)])
        return carry

    lax.fori_loop(0, CHUNKS_PER_W, chunk_body, 0)


@functools.partial(
    pl.kernel,
    mesh=plsc.VectorSubcoreMesh(core_axis_name="c", subcore_axis_name="s"),
    out_type=jax.ShapeDtypeStruct((BATCH * NUM_PAIRS * EMBED_DIM,), jnp.float32),
    scratch_types=[
        pltpu.VMEM((IDX_PER_CHUNK,), jnp.int32),
        pltpu.VMEM((IDX_PER_CHUNK, EMBED_DIM), jnp.float32),
        pltpu.VMEM((OUT_PER_CHUNK,), jnp.float32),
        pltpu.VMEM((FX_PER_CHUNK,), jnp.float32),
        pltpu.VMEM((NUM_FIELDS * NUM_FLOAT_FIELDS * EMBED_DIM,), jnp.float32),
        pltpu.SemaphoreType.DMA,
    ],
    compiler_params=pltpu.CompilerParams(use_tc_tiling_on_sc=False),
)
def _sc_run(tables, idx_all, fxr, ftf, out, idx_v, gbuf, obuf, fxv, ftv, sem):
    _sc_body(tables, idx_all, fxr, ftf, out, idx_v, gbuf, obuf, fxv, ftv, sem)


def kernel(token_x, float_x, token_tables, float_tables):
    tx = token_x.astype(jnp.int32)
    f_off = jnp.arange(NUM_TOKEN_FIELDS, dtype=jnp.int32) * VOCAB_PER_FIELD
    t_off = jnp.arange(NUM_FIELDS, dtype=jnp.int32) * TABLE_ROWS
    idx = tx + f_off[None, :]
    idx_all = idx[:, :, None] + t_off[None, None, :]
    idx_all = idx_all.reshape(CHUNKS, IDX_PER_CHUNK)

    fxr = jnp.broadcast_to(
        float_x.astype(jnp.float32)[:, :, None],
        (BATCH, NUM_FLOAT_FIELDS, L)).reshape(-1)
    ftf = float_tables.astype(jnp.float32).reshape(-1)
    tables2d = token_tables.reshape(NUM_FIELDS * TABLE_ROWS, EMBED_DIM)

    out = _sc_run(tables2d, idx_all, fxr, ftf)
    return out.reshape(BATCH, NUM_PAIRS, EMBED_DIM)


# speedup vs baseline: 1.0030x; 1.0030x over previous
import functools

import jax
import jax.numpy as jnp
from jax import lax
from jax.experimental import pallas as pl
from jax.experimental.pallas import tpu as pltpu
from jax.experimental.pallas import tpu_sc as plsc

NUM_FIELDS = 26
NUM_TOKEN_FIELDS = 20
NUM_FLOAT_FIELDS = 6
EMBED_DIM = 32
BATCH = 1024
VOCAB_PER_FIELD = 5000
TABLE_ROWS = 100000
NUM_PAIRS = (NUM_FIELDS * (NUM_FIELDS - 1)) // 2

L = 16
H = EMBED_DIM // L
NC, NS = 2, 16
NW = NC * NS
R = 4
CHUNKS = BATCH // R
CHUNKS_PER_W = CHUNKS // NW
IDX_PER_CHUNK = R * NUM_TOKEN_FIELDS
ROWS_PER_TABLE = IDX_PER_CHUNK
GBUF_ROWS = NUM_FIELDS * ROWS_PER_TABLE
FX_PER_CHUNK = R * NUM_FLOAT_FIELDS * L


def _pair_index(i, j):
    return 25 * i - (i * (i - 1)) // 2 + (j - i - 1)


def _sc_body(tables, idx_all, fxr, ftf, out, idx_v, gbuf, obuf, fxv, ftv, sem):
    wid = lax.axis_index("s") * NC + lax.axis_index("c")
    pltpu.sync_copy(ftf, ftv)

    def chunk_body(g, carry):
        chunk = wid * CHUNKS_PER_W + g
        pltpu.sync_copy(idx_all.at[chunk], idx_v)
        pltpu.sync_copy(fxr.at[pl.ds(chunk * FX_PER_CHUNK, FX_PER_CHUNK)], fxv)
        copies = [
            pltpu.async_copy(
                tables.at[t].at[idx_v],
                gbuf.at[pl.ds(t * ROWS_PER_TABLE, ROWS_PER_TABLE)],
                sem,
            )
            for t in range(NUM_FIELDS)
        ]
        for c in copies:
            c.wait()

        def tt_i(i, c1):
            def tt_j(j, c2):
                p = 25 * i - (i * (i - 1)) // 2 + (j - i - 1)
                for r in range(R):
                    ra = j * ROWS_PER_TABLE + r * NUM_TOKEN_FIELDS + i
                    rb = i * ROWS_PER_TABLE + r * NUM_TOKEN_FIELDS + j
                    for h in range(H):
                        a = gbuf[ra, pl.ds(h * L, L)]
                        b = gbuf[rb, pl.ds(h * L, L)]
                        obuf[r, p, pl.ds(h * L, L)] = a * b
                return c2

            return lax.fori_loop(i + 1, NUM_TOKEN_FIELDS, tt_j, c1)

        lax.fori_loop(0, NUM_TOKEN_FIELDS, tt_i, 0)

        def tf_i(i, c1):
            p0 = 25 * i - (i * (i - 1)) // 2 + (19 - i)
            for k in range(NUM_FLOAT_FIELDS):
                p = p0 + k
                ft_base = (i * NUM_FLOAT_FIELDS + k) * EMBED_DIM
                for r in range(R):
                    ra = ((NUM_TOKEN_FIELDS + k) * ROWS_PER_TABLE
                          + r * NUM_TOKEN_FIELDS + i)
                    fx = fxv[pl.ds((r * NUM_FLOAT_FIELDS + k) * L, L)]
                    for h in range(H):
                        ft = ftv[pl.ds(ft_base + h * L, L)]
                        a = gbuf[ra, pl.ds(h * L, L)]
                        obuf[r, p, pl.ds(h * L, L)] = a * (fx * ft)
            return c1

        lax.fori_loop(0, NUM_TOKEN_FIELDS, tf_i, 0)

        for a in range(NUM_FLOAT_FIELDS - 1):
            for b in range(a + 1, NUM_FLOAT_FIELDS):
                p = _pair_index(NUM_TOKEN_FIELDS + a, NUM_TOKEN_FIELDS + b)
                fta_base = ((NUM_TOKEN_FIELDS + b) * NUM_FLOAT_FIELDS + a) * EMBED_DIM
                ftb_base = ((NUM_TOKEN_FIELDS + a) * NUM_FLOAT_FIELDS + b) * EMBED_DIM
                for r in range(R):
                    fxa = fxv[pl.ds((r * NUM_FLOAT_FIELDS + a) * L, L)]
                    fxb = fxv[pl.ds((r * NUM_FLOAT_FIELDS + b) * L, L)]
                    for h in range(H):
                        fta = ftv[pl.ds(fta_base + h * L, L)]
                        ftb = ftv[pl.ds(ftb_base + h * L, L)]
                        obuf[r, p, pl.ds(h * L, L)] = (fxa * fta) * (fxb * ftb)

        pltpu.sync_copy(obuf, out.at[pl.ds(chunk * R, R)])
        return carry

    lax.fori_loop(0, CHUNKS_PER_W, chunk_body, 0)


@functools.partial(
    pl.kernel,
    mesh=plsc.VectorSubcoreMesh(core_axis_name="c", subcore_axis_name="s"),
    out_type=jax.ShapeDtypeStruct((BATCH, NUM_PAIRS, EMBED_DIM), jnp.float32),
    scratch_types=[
        pltpu.VMEM((IDX_PER_CHUNK,), jnp.int32),
        pltpu.VMEM((GBUF_ROWS, EMBED_DIM), jnp.float32),
        pltpu.VMEM((R, NUM_PAIRS, EMBED_DIM), jnp.float32),
        pltpu.VMEM((FX_PER_CHUNK,), jnp.float32),
        pltpu.VMEM((NUM_FIELDS * NUM_FLOAT_FIELDS * EMBED_DIM,), jnp.float32),
        pltpu.SemaphoreType.DMA,
    ],
    compiler_params=pltpu.CompilerParams(use_tc_tiling_on_sc=False),
)
def _sc_run(tables, idx_all, fxr, ftf, out, idx_v, gbuf, obuf, fxv, ftv, sem):
    _sc_body(tables, idx_all, fxr, ftf, out, idx_v, gbuf, obuf, fxv, ftv, sem)


def kernel(token_x, float_x, token_tables, float_tables):
    tx = token_x.astype(jnp.int32)
    f_off = jnp.arange(NUM_TOKEN_FIELDS, dtype=jnp.int32) * VOCAB_PER_FIELD
    idx_all = (tx + f_off[None, :]).reshape(CHUNKS, IDX_PER_CHUNK)

    fxr = jnp.broadcast_to(
        float_x.astype(jnp.float32)[:, :, None],
        (BATCH, NUM_FLOAT_FIELDS, L)).reshape(-1)
    ftf = float_tables.astype(jnp.float32).reshape(-1)

    return _sc_run(token_tables, idx_all, fxr, ftf)
